# baseline (device time: 10667 ns/iter reference)
import jax
import jax.numpy as jnp
from jax import lax
from jax.experimental import pallas as pl
from jax.experimental.pallas import tpu as pltpu

N_DEV = 4
K = 8
LANES = 128
HALF = 2

_BATCHER8 = [
    (0, 1), (2, 3), (4, 5), (6, 7),
    (0, 2), (1, 3), (4, 6), (5, 7),
    (1, 2), (5, 6),
    (0, 4), (1, 5), (2, 6), (3, 7),
    (2, 4), (3, 5),
    (1, 2), (3, 4), (5, 6),
]

_NEG = float("-inf")
_BIG = 30000.0


def _topk_desc(vals, k):
    m, n = vals.shape
    col = lax.broadcasted_iota(jnp.int32, (m, n), 1).astype(jnp.bfloat16)
    tops = []
    for t in range(k):
        mx = jnp.max(vals, axis=1, keepdims=True)
        tops.append(mx)
        if t < k - 1:
            first = jnp.min(
                jnp.where(vals == mx, col, _BIG), axis=1, keepdims=True
            )
            vals = jnp.where(col == first, _NEG, vals)
    return jnp.concatenate(tops, axis=1)


def _local_topk_slab(xb, k):
    m = xb.shape[0]
    slabs = [xb[:, g * LANES:(g + 1) * LANES] for g in range(8)]
    for i, j in _BATCHER8:
        hi = jnp.maximum(slabs[i], slabs[j])
        lo = jnp.minimum(slabs[i], slabs[j])
        slabs[i], slabs[j] = hi, lo

    col = lax.broadcasted_iota(jnp.int32, (m, LANES), 1).astype(jnp.bfloat16)
    tops = []
    for t in range(k):
        mx = jnp.max(slabs[0], axis=1, keepdims=True)
        tops.append(mx)
        if t < k - 1:
            first = jnp.min(
                jnp.where(slabs[0] == mx, col, _BIG), axis=1, keepdims=True
            )
            hit = col == first
            for j in range(7):
                slabs[j] = jnp.where(hit, slabs[j + 1], slabs[j])
            slabs[7] = jnp.where(hit, _NEG, slabs[7])
    return jnp.concatenate(tops, axis=1)


def kernel(x):
    m, n = x.shape
    rows = m // HALF

    def body(x_ref, out_ref, cand_ref, send_sems, recv_sems):
        my = lax.axis_index("i")

        barrier = pltpu.get_barrier_semaphore()
        for p in range(1, N_DEV):
            pl.semaphore_signal(
                barrier,
                inc=1,
                device_id=((my + p) % N_DEV,),
                device_id_type=pl.DeviceIdType.MESH,
            )

        def send_half(h):
            rdmas = []
            r0 = h * rows
            for p in range(1, N_DEV):
                rdma = pltpu.make_async_remote_copy(
                    src_ref=cand_ref.at[0, pl.ds(r0, rows)],
                    dst_ref=cand_ref.at[N_DEV - p, pl.ds(r0, rows)],
                    send_sem=send_sems.at[h, p - 1],
                    recv_sem=recv_sems.at[h, p - 1],
                    device_id=((my + p) % N_DEV,),
                    device_id_type=pl.DeviceIdType.MESH,
                )
                rdma.start()
                rdmas.append(rdma)
            return rdmas

        def merge_half(h):
            r0 = h * rows
            allc = jnp.concatenate(
                [cand_ref[i, pl.ds(r0, rows), :] for i in range(N_DEV)],
                axis=1,
            )
            out_ref[pl.ds(r0, rows), :] = _topk_desc(allc, K)

        xb1 = x_ref[0:rows, :].astype(jnp.bfloat16)
        cand_ref[0, 0:rows, :] = _local_topk_slab(xb1, K)
        pl.semaphore_wait(barrier, N_DEV - 1)
        rdmas1 = send_half(0)

        xb2 = x_ref[rows:, :].astype(jnp.bfloat16)
        cand_ref[0, rows:, :] = _local_topk_slab(xb2, K)
        rdmas2 = send_half(1)

        for rdma in rdmas1:
            rdma.wait()
        merge_half(0)
        for rdma in rdmas2:
            rdma.wait()
        merge_half(1)

    return pl.pallas_call(
        body,
        out_shape=jax.ShapeDtypeStruct((m, K), jnp.bfloat16),
        in_specs=[pl.BlockSpec(memory_space=pltpu.VMEM)],
        out_specs=pl.BlockSpec(memory_space=pltpu.VMEM),
        scratch_shapes=[
            pltpu.VMEM((N_DEV, m, K), jnp.bfloat16),
            pltpu.SemaphoreType.DMA((HALF, N_DEV - 1)),
            pltpu.SemaphoreType.DMA((HALF, N_DEV - 1)),
        ],
        compiler_params=pltpu.CompilerParams(collective_id=0),
    )(x)


# device time: 9735 ns/iter; 1.0957x vs baseline; 1.0957x over previous
import jax
import jax.numpy as jnp
from jax import lax
from jax.experimental import pallas as pl
from jax.experimental.pallas import tpu as pltpu

N_DEV = 4
K = 8
LANES = 128

_BATCHER8 = [
    (0, 1), (2, 3), (4, 5), (6, 7),
    (0, 2), (1, 3), (4, 6), (5, 7),
    (1, 2), (5, 6),
    (0, 4), (1, 5), (2, 6), (3, 7),
    (2, 4), (3, 5),
    (1, 2), (3, 4), (5, 6),
]

_NEG = float("-inf")
_BIG = 30000.0


def _topk_desc(vals, k):
    m, n = vals.shape
    col = lax.broadcasted_iota(jnp.int32, (m, n), 1).astype(jnp.bfloat16)
    tops = []
    for t in range(k):
        mx = jnp.max(vals, axis=1, keepdims=True)
        tops.append(mx)
        if t < k - 1:
            first = jnp.min(
                jnp.where(vals == mx, col, _BIG), axis=1, keepdims=True
            )
            vals = jnp.where(col == first, _NEG, vals)
    return jnp.concatenate(tops, axis=1)


def _local_topk_slab(xb, k):
    m = xb.shape[0]
    slabs = [xb[:, g * LANES:(g + 1) * LANES] for g in range(8)]
    for i, j in _BATCHER8:
        hi = jnp.maximum(slabs[i], slabs[j])
        lo = jnp.minimum(slabs[i], slabs[j])
        slabs[i], slabs[j] = hi, lo

    col = lax.broadcasted_iota(jnp.int32, (m, LANES), 1).astype(jnp.bfloat16)
    tops = []
    for t in range(k):
        mx = jnp.max(slabs[0], axis=1, keepdims=True)
        tops.append(mx)
        if t < k - 1:
            first = jnp.min(
                jnp.where(slabs[0] == mx, col, _BIG), axis=1, keepdims=True
            )
            hit = col == first
            for j in range(7):
                slabs[j] = jnp.where(hit, slabs[j + 1], slabs[j])
            slabs[7] = jnp.where(hit, _NEG, slabs[7])
    return jnp.concatenate(tops, axis=1)


def kernel(x):
    m, n = x.shape

    def body(x_ref, out_ref, cand_ref, send_sems, recv_sems):
        my = lax.axis_index("i")

        barrier = pltpu.get_barrier_semaphore()
        for p in range(1, N_DEV):
            pl.semaphore_signal(
                barrier,
                inc=1,
                device_id=((my + p) % N_DEV,),
                device_id_type=pl.DeviceIdType.MESH,
            )

        xb = x_ref[:, :].astype(jnp.bfloat16)
        cand_ref[0, :, :] = _local_topk_slab(xb, K)

        pl.semaphore_wait(barrier, N_DEV - 1)

        rdmas = []
        for p in range(1, N_DEV):
            rdma = pltpu.make_async_remote_copy(
                src_ref=cand_ref.at[0],
                dst_ref=cand_ref.at[N_DEV - p],
                send_sem=send_sems.at[p - 1],
                recv_sem=recv_sems.at[p - 1],
                device_id=((my + p) % N_DEV,),
                device_id_type=pl.DeviceIdType.MESH,
            )
            rdma.start()
            rdmas.append(rdma)
        for rdma in rdmas:
            rdma.wait()

        allc = jnp.concatenate(
            [cand_ref[i, :, :] for i in range(N_DEV)], axis=1
        )
        out_ref[:, :] = _topk_desc(allc, K)

    return pl.pallas_call(
        body,
        out_shape=jax.ShapeDtypeStruct((m, K), jnp.bfloat16),
        in_specs=[pl.BlockSpec(memory_space=pltpu.VMEM)],
        out_specs=pl.BlockSpec(memory_space=pltpu.VMEM),
        scratch_shapes=[
            pltpu.VMEM((N_DEV, m, K), jnp.bfloat16),
            pltpu.SemaphoreType.DMA((N_DEV - 1,)),
            pltpu.SemaphoreType.DMA((N_DEV - 1,)),
        ],
        compiler_params=pltpu.CompilerParams(collective_id=0),
    )(x)
